# baseline (speedup 1.0000x reference)
import jax
import jax.numpy as jnp
from jax.experimental import pallas as pl
from jax.experimental.pallas import tpu as pltpu


def _gcn_kernel(x_ref, a_any, w1_ref, w2_ref, w3_ref, b1_ref, b2_ref, b3_ref,
                wd1_ref, bd1_ref, wd2_ref, bd2_ref, wo_ref, bo_ref,
                out_ref, a_sc, y_sc, h_sc, stage, sem, *, rb, nb):
    s = pl.program_id(1)
    b = pl.program_id(0)
    n_b = pl.num_programs(0)

    def copy(bb, blk, slot):
        return pltpu.make_async_copy(
            a_any.at[bb, pl.ds(blk * rb, rb), :],
            stage.at[slot],
            sem.at[slot])

    @pl.when(s == 0)
    def _():
        y_sc[...] = jnp.dot(x_ref[0].astype(jnp.bfloat16),
                            w1_ref[...].astype(jnp.bfloat16),
                            preferred_element_type=jnp.float32).astype(jnp.bfloat16)

    @pl.when(s > 0)
    def _():
        wt = jnp.where(s == 1, w2_ref[...], w3_ref[...])
        y_sc[...] = jnp.dot(h_sc[...].astype(jnp.bfloat16),
                            wt.astype(jnp.bfloat16),
                            preferred_element_type=jnp.float32).astype(jnp.bfloat16)

    bias = jnp.where(s == 0, b1_ref[...],
                     jnp.where(s == 1, b2_ref[...], b3_ref[...]))

    @pl.when(jnp.logical_and(s == 0, b == 0))
    def _():
        copy(b, 0, 0).start()
        copy(b, 1, 1).start()

        def body(i, carry):
            slot = jax.lax.rem(i, 2)
            copy(b, i, slot).wait()
            a_bf = stage[slot].astype(jnp.bfloat16)
            a_sc[pl.ds(i * rb, rb), :] = a_bf
            z = jnp.dot(a_bf, y_sc[...], preferred_element_type=jnp.float32)
            h_sc[pl.ds(i * rb, rb), :] = jnp.maximum(z + bias, 0.0)

            @pl.when(i + 2 < nb)
            def _():
                copy(b, i + 2, slot).start()
            return carry

        jax.lax.fori_loop(0, nb, body, 0, unroll=2)

    half = nb // 2

    @pl.when(jnp.logical_and(s == 0, b > 0))
    def _():
        def body(i, carry):
            @pl.when(i < half)
            def _():
                slot = jax.lax.rem(i, 2)
                copy(b, half + i, slot).wait()
                a_sc[pl.ds((half + i) * rb, rb), :] = stage[slot].astype(jnp.bfloat16)

                @pl.when(i + half + 2 < nb)
                def _():
                    copy(b, half + i + 2, slot).start()

            a_bf = a_sc[pl.ds(i * rb, rb), :]
            z = jnp.dot(a_bf, y_sc[...], preferred_element_type=jnp.float32)
            h_sc[pl.ds(i * rb, rb), :] = jnp.maximum(z + bias, 0.0)
            return carry

        jax.lax.fori_loop(0, nb, body, 0, unroll=2)

    @pl.when(s == 1)
    def _():
        @pl.when(b < n_b - 1)
        def _():
            copy(b + 1, 0, 0).start()
            copy(b + 1, 1, 1).start()

        def body(i, carry):
            a_bf = a_sc[pl.ds(i * rb, rb), :]
            z = jnp.dot(a_bf, y_sc[...], preferred_element_type=jnp.float32)
            h_sc[pl.ds(i * rb, rb), :] = jnp.maximum(z + bias, 0.0)
            return carry

        jax.lax.fori_loop(0, nb, body, 0, unroll=2)

    @pl.when(s == 2)
    def _():
        def body(i, carry):
            a_bf = a_sc[pl.ds(i * rb, rb), :]
            z = jnp.dot(a_bf, y_sc[...], preferred_element_type=jnp.float32)
            h_sc[pl.ds(i * rb, rb), :] = jnp.maximum(z + bias, 0.0)

            @pl.when(jnp.logical_and(b < n_b - 1, i >= half))
            def _():
                slot = jax.lax.rem(i, 2)
                copy(b + 1, i - half, slot).wait()
                a_sc[pl.ds((i - half) * rb, rb), :] = stage[slot].astype(jnp.bfloat16)
                copy(b + 1, i - half + 2, slot).start()
            return carry

        jax.lax.fori_loop(0, nb, body, 0, unroll=2)

    @pl.when(s == 2)
    def _():
        p = jnp.mean(h_sc[...], axis=0, keepdims=True)
        p8 = jnp.broadcast_to(p, (8, p.shape[1])).astype(jnp.bfloat16)
        z1 = jnp.maximum(
            jnp.dot(p8, wd1_ref[...].astype(jnp.bfloat16),
                    preferred_element_type=jnp.float32) + bd1_ref[...], 0.0)
        z2 = jnp.maximum(
            jnp.dot(z1.astype(jnp.bfloat16), wd2_ref[...].astype(jnp.bfloat16),
                    preferred_element_type=jnp.float32) + bd2_ref[...], 0.0)
        logits = jnp.dot(z2.astype(jnp.bfloat16), wo_ref[...].astype(jnp.bfloat16),
                         preferred_element_type=jnp.float32) + bo_ref[...]
        m = jnp.max(logits, axis=-1, keepdims=True)
        e = jnp.exp(logits - m)
        sm = e / jnp.sum(e, axis=-1, keepdims=True)
        out_ref[pl.ds(b, 1), :] = sm[0:1, :]


def kernel(x, a, W1, b1, W2, b2, W3, b3, Wd1, bd1, Wd2, bd2, Wo, bo):
    B, N, F = x.shape
    H = W1.shape[1]
    L = Wo.shape[1]
    RB = 512
    NB = N // RB

    grid = (B, 3)

    def full(arr):
        nd = arr.ndim
        return pl.BlockSpec(arr.shape, lambda b, s: (0,) * nd)

    b1r, b2r, b3r = b1.reshape(1, -1), b2.reshape(1, -1), b3.reshape(1, -1)
    bd1r, bd2r, bor = bd1.reshape(1, -1), bd2.reshape(1, -1), bo.reshape(1, -1)

    in_specs = [
        pl.BlockSpec((1, N, F), lambda b, s: (b, 0, 0)),
        pl.BlockSpec(memory_space=pl.ANY),
        full(W1), full(W2), full(W3),
        full(b1r), full(b2r), full(b3r),
        full(Wd1), full(bd1r), full(Wd2), full(bd2r), full(Wo), full(bor),
    ]

    out = pl.pallas_call(
        lambda *refs: _gcn_kernel(*refs, rb=RB, nb=NB),
        grid=grid,
        in_specs=in_specs,
        out_specs=pl.BlockSpec((B, L), lambda b, s: (0, 0)),
        out_shape=jax.ShapeDtypeStruct((B, L), jnp.float32),
        scratch_shapes=[
            pltpu.VMEM((N, N), jnp.bfloat16),
            pltpu.VMEM((N, H), jnp.bfloat16),
            pltpu.VMEM((N, H), jnp.float32),
            pltpu.VMEM((2, RB, N), jnp.float32),
            pltpu.SemaphoreType.DMA((2,)),
        ],
        compiler_params=pltpu.CompilerParams(
            dimension_semantics=("arbitrary", "arbitrary"),
        ),
    )(x, a, W1, W2, W3, b1r, b2r, b3r, Wd1, bd1r, Wd2, bd2r, Wo, bor)
    return out


# speedup vs baseline: 1.2563x; 1.2563x over previous
import jax
import jax.numpy as jnp
from jax.experimental import pallas as pl
from jax.experimental.pallas import tpu as pltpu


def _gcn_kernel(x_ref, a_any, w1_ref, w2_ref, w3_ref, b1_ref, b2_ref, b3_ref,
                wd1_ref, bd1_ref, wd2_ref, bd2_ref, wo_ref, bo_ref,
                out_ref, a_sc, y_sc, h_sc, stage, sem, *, rb, nb):
    s = pl.program_id(1)
    b = pl.program_id(0)
    n_b = pl.num_programs(0)

    def copy(bb, blk, slot):
        return pltpu.make_async_copy(
            a_any.at[bb, pl.ds(blk * rb, rb), :],
            stage.at[slot],
            sem.at[slot])

    @pl.when(s == 0)
    def _():
        y_sc[...] = jnp.dot(x_ref[0].astype(jnp.bfloat16),
                            w1_ref[...].astype(jnp.bfloat16),
                            preferred_element_type=jnp.float32).astype(jnp.bfloat16)

    @pl.when(s > 0)
    def _():
        wt = jnp.where(s == 1, w2_ref[...], w3_ref[...])
        y_sc[...] = jnp.dot(h_sc[...].astype(jnp.bfloat16),
                            wt.astype(jnp.bfloat16),
                            preferred_element_type=jnp.float32).astype(jnp.bfloat16)

    bias = jnp.where(s == 0, b1_ref[...],
                     jnp.where(s == 1, b2_ref[...], b3_ref[...]))

    @pl.when(jnp.logical_and(s == 0, b == 0))
    def _():
        copy(b, 0, 0).start()
        copy(b, 1, 1).start()

        def body(i, carry):
            slot = jax.lax.rem(i, 2)
            copy(b, i, slot).wait()
            a_bf = stage[slot].astype(jnp.bfloat16)
            a_sc[pl.ds(i * rb, rb), :] = a_bf
            z = jnp.dot(a_bf, y_sc[...], preferred_element_type=jnp.float32)
            h_sc[pl.ds(i * rb, rb), :] = jnp.maximum(z + bias, 0.0)

            @pl.when(i + 2 < nb)
            def _():
                copy(b, i + 2, slot).start()
            return carry

        jax.lax.fori_loop(0, nb, body, 0)

    half = nb // 2

    @pl.when(jnp.logical_and(s == 0, b > 0))
    def _():
        def body(i, carry):
            @pl.when(i < half)
            def _():
                slot = jax.lax.rem(i, 2)
                copy(b, half + i, slot).wait()
                a_sc[pl.ds((half + i) * rb, rb), :] = stage[slot].astype(jnp.bfloat16)

                @pl.when(i + half + 2 < nb)
                def _():
                    copy(b, half + i + 2, slot).start()

            a_bf = a_sc[pl.ds(i * rb, rb), :]
            z = jnp.dot(a_bf, y_sc[...], preferred_element_type=jnp.float32)
            h_sc[pl.ds(i * rb, rb), :] = jnp.maximum(z + bias, 0.0)
            return carry

        jax.lax.fori_loop(0, nb, body, 0)

    @pl.when(s == 1)
    def _():
        @pl.when(b < n_b - 1)
        def _():
            copy(b + 1, 0, 0).start()
            copy(b + 1, 1, 1).start()

        def body(i, carry):
            a_bf = a_sc[pl.ds(i * rb, rb), :]
            z = jnp.dot(a_bf, y_sc[...], preferred_element_type=jnp.float32)
            h_sc[pl.ds(i * rb, rb), :] = jnp.maximum(z + bias, 0.0)
            return carry

        jax.lax.fori_loop(0, nb, body, 0)

    @pl.when(s == 2)
    def _():
        def body(i, carry):
            a_bf = a_sc[pl.ds(i * rb, rb), :]
            z = jnp.dot(a_bf, y_sc[...], preferred_element_type=jnp.float32)
            h_sc[pl.ds(i * rb, rb), :] = jnp.maximum(z + bias, 0.0)

            @pl.when(jnp.logical_and(b < n_b - 1, i >= half))
            def _():
                slot = jax.lax.rem(i, 2)
                copy(b + 1, i - half, slot).wait()
                a_sc[pl.ds((i - half) * rb, rb), :] = stage[slot].astype(jnp.bfloat16)
                copy(b + 1, i - half + 2, slot).start()
            return carry

        jax.lax.fori_loop(0, nb, body, 0)

    @pl.when(s == 2)
    def _():
        p = jnp.mean(h_sc[...], axis=0, keepdims=True)
        p8 = jnp.broadcast_to(p, (8, p.shape[1])).astype(jnp.bfloat16)
        z1 = jnp.maximum(
            jnp.dot(p8, wd1_ref[...].astype(jnp.bfloat16),
                    preferred_element_type=jnp.float32) + bd1_ref[...], 0.0)
        z2 = jnp.maximum(
            jnp.dot(z1.astype(jnp.bfloat16), wd2_ref[...].astype(jnp.bfloat16),
                    preferred_element_type=jnp.float32) + bd2_ref[...], 0.0)
        logits = jnp.dot(z2.astype(jnp.bfloat16), wo_ref[...].astype(jnp.bfloat16),
                         preferred_element_type=jnp.float32) + bo_ref[...]
        m = jnp.max(logits, axis=-1, keepdims=True)
        e = jnp.exp(logits - m)
        sm = e / jnp.sum(e, axis=-1, keepdims=True)
        out_ref[pl.ds(b, 1), :] = sm[0:1, :]


def kernel(x, a, W1, b1, W2, b2, W3, b3, Wd1, bd1, Wd2, bd2, Wo, bo):
    B, N, F = x.shape
    H = W1.shape[1]
    L = Wo.shape[1]
    RB = 512
    NB = N // RB

    grid = (B, 3)

    def full(arr):
        nd = arr.ndim
        return pl.BlockSpec(arr.shape, lambda b, s: (0,) * nd)

    b1r, b2r, b3r = b1.reshape(1, -1), b2.reshape(1, -1), b3.reshape(1, -1)
    bd1r, bd2r, bor = bd1.reshape(1, -1), bd2.reshape(1, -1), bo.reshape(1, -1)

    in_specs = [
        pl.BlockSpec((1, N, F), lambda b, s: (b, 0, 0)),
        pl.BlockSpec(memory_space=pl.ANY),
        full(W1), full(W2), full(W3),
        full(b1r), full(b2r), full(b3r),
        full(Wd1), full(bd1r), full(Wd2), full(bd2r), full(Wo), full(bor),
    ]

    out = pl.pallas_call(
        lambda *refs: _gcn_kernel(*refs, rb=RB, nb=NB),
        grid=grid,
        in_specs=in_specs,
        out_specs=pl.BlockSpec((B, L), lambda b, s: (0, 0)),
        out_shape=jax.ShapeDtypeStruct((B, L), jnp.float32),
        scratch_shapes=[
            pltpu.VMEM((N, N), jnp.bfloat16),
            pltpu.VMEM((N, H), jnp.bfloat16),
            pltpu.VMEM((N, H), jnp.float32),
            pltpu.VMEM((2, RB, N), jnp.float32),
            pltpu.SemaphoreType.DMA((2,)),
        ],
        compiler_params=pltpu.CompilerParams(
            dimension_semantics=("arbitrary", "arbitrary"),
        ),
    )(x, a, W1, W2, W3, b1r, b2r, b3r, Wd1, bd1r, Wd2, bd2r, Wo, bor)
    return out
